# baseline (speedup 1.0000x reference)
import functools

import jax
import jax.numpy as jnp
from jax import lax
from jax.experimental import pallas as pl
from jax.experimental.pallas import tpu as pltpu
from jax.experimental.pallas import tpu_sc as plsc

_LANES = 16
_GROUP = 4


@functools.lru_cache(maxsize=None)
def _build_sc_embed(n_rows: int, vocab: int, hidden: int):
    info = plsc.get_sparse_core_info()
    nc, ns = info.num_cores, info.num_subcores
    nw = nc * ns
    combos = 2 ** _GROUP
    assert vocab == 2 and hidden % _LANES == 0
    assert n_rows % (nw * _GROUP * _LANES) == 0
    rows_per_w = n_rows // nw
    groups_per_w = rows_per_w // _GROUP

    mesh = plsc.VectorSubcoreMesh(core_axis_name="c", subcore_axis_name="s")

    @functools.partial(
        pl.kernel,
        out_type=jax.ShapeDtypeStruct((n_rows, hidden), jnp.float32),
        mesh=mesh,
        scratch_types=[
            pltpu.VMEM((vocab, hidden), jnp.float32),
            pltpu.VMEM((rows_per_w,), jnp.int32),
            pltpu.VMEM((combos, _GROUP, hidden), jnp.float32),
            pltpu.SemaphoreType.DMA,
        ],
    )
    def embed(x_hbm, table_hbm, out_hbm, table_v, idx_v, combo_v, sem):
        wid = lax.axis_index("s") * nc + lax.axis_index("c")
        base = wid * rows_per_w
        pltpu.sync_copy(table_hbm, table_v)
        pltpu.sync_copy(x_hbm.at[pl.ds(base, rows_per_w)], idx_v)

        def build(cc, carry):
            col = cc * _LANES
            t0 = table_v[0, pl.ds(col, _LANES)]
            t1 = table_v[1, pl.ds(col, _LANES)]
            for c in range(combos):
                for j in range(_GROUP):
                    combo_v[c, j, pl.ds(col, _LANES)] = (
                        t1 if (c >> j) & 1 else t0
                    )
            return carry

        lax.fori_loop(0, hidden // _LANES, build, 0)

        def issue(g, carry):
            v = idx_v[pl.ds(g * _LANES, _LANES)]
            for t in range(_LANES // _GROUP):
                p = v[t * _GROUP]
                for j in range(1, _GROUP):
                    p = p + v[t * _GROUP + j] * (2 ** j)
                row = base + g * _LANES + t * _GROUP
                pltpu.make_async_copy(
                    combo_v.at[p], out_hbm.at[pl.ds(row, _GROUP)], sem
                ).start()
            return carry

        lax.fori_loop(0, rows_per_w // _LANES, issue, 0)

        pltpu.make_async_copy(
            out_hbm.at[pl.ds(base, rows_per_w)],
            out_hbm.at[pl.ds(base, rows_per_w)],
            sem,
        ).wait()

    return embed


def kernel(x, table):
    b, s = x.shape
    n = b * s
    xf = x.reshape(n).astype(jnp.int32)
    out_flat = _build_sc_embed(n, table.shape[0], table.shape[1])(xf, table)
    return out_flat.reshape(b, s, table.shape[1])


# speedup vs baseline: 1.0153x; 1.0153x over previous
import functools

import jax
import jax.numpy as jnp
from jax import lax
from jax.experimental import pallas as pl
from jax.experimental.pallas import tpu as pltpu
from jax.experimental.pallas import tpu_sc as plsc

_LANES = 16


@functools.lru_cache(maxsize=None)
def _build_sc_embed(n_rows: int, vocab: int, hidden: int):
    info = plsc.get_sparse_core_info()
    nc, ns = info.num_cores, info.num_subcores
    nw = nc * ns
    assert n_rows % (nw * _LANES) == 0
    rows_per_w = n_rows // nw

    mesh = plsc.VectorSubcoreMesh(core_axis_name="c", subcore_axis_name="s")

    @functools.partial(
        pl.kernel,
        out_type=jax.ShapeDtypeStruct((n_rows, hidden), jnp.float32),
        mesh=mesh,
        scratch_types=[
            pltpu.VMEM((vocab, hidden), jnp.float32),
            pltpu.VMEM((rows_per_w,), jnp.int32),
            pltpu.SemaphoreType.DMA,
            pltpu.SemaphoreType.DMA,
            pltpu.SemaphoreType.DMA,
        ],
    )
    def embed(x_hbm, table_hbm, out_hbm, table_v, idx_v, sem_in, sem_a, sem_b):
        wid = lax.axis_index("s") * nc + lax.axis_index("c")
        base = wid * rows_per_w
        tab_cp = pltpu.make_async_copy(table_hbm, table_v, sem_in)
        idx_cp = pltpu.make_async_copy(
            x_hbm.at[pl.ds(base, rows_per_w)], idx_v, sem_in
        )
        tab_cp.start()
        idx_cp.start()
        tab_cp.wait()
        idx_cp.wait()

        def body(g, carry):
            row0 = g * _LANES
            xv = idx_v[pl.ds(row0, _LANES)]
            for j in range(_LANES):
                pltpu.make_async_copy(
                    table_v.at[xv[j]],
                    out_hbm.at[base + row0 + j],
                    sem_a if j % 2 == 0 else sem_b,
                ).start()
            return carry

        lax.fori_loop(0, rows_per_w // _LANES, body, 0)

        half = out_hbm.at[pl.ds(base, rows_per_w // 2)]
        pltpu.make_async_copy(half, half, sem_a).wait()
        pltpu.make_async_copy(half, half, sem_b).wait()

    return embed


def kernel(x, table):
    b, s = x.shape
    n = b * s
    xf = x.reshape(n).astype(jnp.int32)
    out_flat = _build_sc_embed(n, table.shape[0], table.shape[1])(xf, table)
    return out_flat.reshape(b, s, table.shape[1])
